# baseline (speedup 1.0000x reference)
import functools

import jax
import jax.numpy as jnp
from jax import lax
from jax.experimental import pallas as pl
from jax.experimental.pallas import tpu as pltpu
from jax.experimental.pallas import tpu_sc as plsc

D = 64
L = 200
B = 16384
IDXW = 128
STEP_IROWS = 5
ROWS_PER_STEP = STEP_IROWS * IDXW
NCHUNK = D // 16

_info = plsc.get_sparse_core_info()
NC, NS = _info.num_cores, _info.num_subcores
NW = NC * NS


def _sc_partial_sums(ids2d, table):
    irows_per_w = ids2d.shape[0] // NW
    nsteps = irows_per_w // STEP_IROWS
    mesh = plsc.VectorSubcoreMesh(core_axis_name="c", subcore_axis_name="s")

    @functools.partial(
        pl.kernel,
        mesh=mesh,
        out_type=jax.ShapeDtypeStruct((NW, L, D), jnp.float32),
        compiler_params=pltpu.CompilerParams(use_tc_tiling_on_sc=False),
        scratch_types=[
            pltpu.VMEM((STEP_IROWS, IDXW), jnp.int32),
            pltpu.VMEM((STEP_IROWS, IDXW), jnp.int32),
            pltpu.VMEM((ROWS_PER_STEP, D), jnp.float32),
            pltpu.VMEM((ROWS_PER_STEP, D), jnp.float32),
            pltpu.VMEM((L, D), jnp.float32),
            pltpu.SemaphoreType.DMA,
            pltpu.SemaphoreType.DMA,
            pltpu.SemaphoreType.DMA,
        ],
    )
    def k(ids_hbm, table_hbm, out_hbm, idx0, idx1, buf0, buf1, acc,
          sem0, sem1, isem):
        wid = lax.axis_index("s") * NC + lax.axis_index("c")
        base_irow = wid * irows_per_w
        idxbufs = (idx0, idx1)
        bufs = (buf0, buf1)
        sems = (sem0, sem1)

        def zero_body(l, _):
            for c in range(NCHUNK):
                acc[l, pl.ds(c * 16, 16)] = jnp.zeros((16,), jnp.float32)
            return 0
        lax.fori_loop(0, L, zero_body, 0)

        def idx_fetch(g, slot):
            irow = base_irow + g * STEP_IROWS
            pltpu.async_copy(ids_hbm.at[pl.ds(irow, STEP_IROWS)],
                             idxbufs[slot], isem)

        def idx_wait(slot):
            pltpu.make_async_copy(ids_hbm.at[pl.ds(0, STEP_IROWS)],
                                  idxbufs[slot], isem).wait()

        def fire(slot):
            for j in range(STEP_IROWS):
                pltpu.async_copy(
                    table_hbm.at[idxbufs[slot].at[j]],
                    bufs[slot].at[pl.ds(j * IDXW, IDXW)],
                    sems[slot],
                )

        def drain(slot):
            pltpu.make_async_copy(
                table_hbm.at[pl.ds(0, ROWS_PER_STEP)], bufs[slot], sems[slot]
            ).wait()

        def accumulate(slot, l0):
            buf = bufs[slot]
            def body(j, l):
                for c in range(NCHUNK):
                    sl = pl.ds(c * 16, 16)
                    plsc.addupdate(acc.at[l, sl], buf[j, sl])
                nl = l + 1
                return jnp.where(nl == L, 0, nl)
            lax.fori_loop(0, ROWS_PER_STEP, body, l0)

        idx_fetch(0, 0)
        idx_wait(0)
        fire(0)
        idx_fetch(1, 1)

        def phase(g, slot, nslot, fetch_ahead):
            idx_wait(nslot)
            fire(nslot)
            drain(slot)
            if fetch_ahead:
                idx_fetch(g + 2, slot)
            accumulate(slot, lax.rem(g * ROWS_PER_STEP, L))

        def main_body(g2, _):
            g = g2 * 2
            phase(g, 0, 1, True)
            phase(g + 1, 1, 0, True)
            return 0
        lax.fori_loop(0, nsteps // 2 - 1, main_body, 0)

        phase(nsteps - 2, 0, 1, False)
        drain(1)
        accumulate(1, lax.rem((nsteps - 1) * ROWS_PER_STEP, L))

        pltpu.sync_copy(acc, out_hbm.at[wid])

    return k(ids2d, table)


def _mlp(partials, W1, b1, W2, b2, W3, b3):
    def body(p_ref, w1_ref, b1_ref, w2_ref, b2_ref, w3_ref, b3_ref, o_ref):
        s = jnp.sum(p_ref[...], axis=0) * (1.0 / B)
        h = jnp.maximum(
            jnp.dot(s, w1_ref[...], preferred_element_type=jnp.float32)
            + b1_ref[...], 0.0)
        h = jnp.maximum(
            jnp.dot(h, w2_ref[...], preferred_element_type=jnp.float32)
            + b2_ref[...], 0.0)
        o_ref[...] = (
            jnp.dot(h, w3_ref[...], preferred_element_type=jnp.float32)
            + b3_ref[...])

    return pl.pallas_call(
        body,
        out_shape=jax.ShapeDtypeStruct((L, D), jnp.float32),
    )(partials, W1, b1.reshape(1, D), W2, b2.reshape(1, D), W3,
      b3.reshape(1, D))


def kernel(tag_ids, table, W1, b1, W2, b2, W3, b3):
    ids2d = tag_ids.astype(jnp.int32).reshape(-1, IDXW)
    partials = _sc_partial_sums(ids2d, table)
    return _mlp(partials, W1, b1, W2, b2, W3, b3)


# speedup vs baseline: 2.1756x; 2.1756x over previous
import functools

import jax
import jax.numpy as jnp
from jax import lax
from jax.experimental import pallas as pl
from jax.experimental.pallas import tpu as pltpu
from jax.experimental.pallas import tpu_sc as plsc

D = 64
L = 200
B = 16384
IDXW = 128
STEP_IROWS = 5
ROWS_PER_STEP = STEP_IROWS * IDXW
NCHUNK = D // 16

_info = plsc.get_sparse_core_info()
NC, NS = _info.num_cores, _info.num_subcores
NW = NC * NS


def _sc_partial_sums(ids2d, table):
    irows_per_w = ids2d.shape[0] // NW
    nsteps = irows_per_w // STEP_IROWS
    irows_per_l = B // IDXW
    mesh = plsc.VectorSubcoreMesh(core_axis_name="c", subcore_axis_name="s")

    @functools.partial(
        pl.kernel,
        mesh=mesh,
        out_type=jax.ShapeDtypeStruct((NW, L, D), jnp.float32),
        compiler_params=pltpu.CompilerParams(use_tc_tiling_on_sc=False),
        scratch_types=[
            pltpu.VMEM((STEP_IROWS, IDXW), jnp.int32),
            pltpu.VMEM((STEP_IROWS, IDXW), jnp.int32),
            pltpu.VMEM((ROWS_PER_STEP, D), jnp.float32),
            pltpu.VMEM((ROWS_PER_STEP, D), jnp.float32),
            pltpu.VMEM((L, D), jnp.float32),
            pltpu.SemaphoreType.DMA,
            pltpu.SemaphoreType.DMA,
            pltpu.SemaphoreType.DMA,
        ],
    )
    def k(ids_hbm, table_hbm, out_hbm, idx0, idx1, buf0, buf1, acc,
          sem0, sem1, isem):
        wid = lax.axis_index("s") * NC + lax.axis_index("c")
        base_irow = wid * irows_per_w
        idxbufs = (idx0, idx1)
        bufs = (buf0, buf1)
        sems = (sem0, sem1)

        def zero_body(l, _):
            for c in range(NCHUNK):
                acc[l, pl.ds(c * 16, 16)] = jnp.zeros((16,), jnp.float32)
            return 0
        lax.fori_loop(0, L, zero_body, 0)

        def idx_fetch(g, slot):
            irow = base_irow + g * STEP_IROWS
            pltpu.async_copy(ids_hbm.at[pl.ds(irow, STEP_IROWS)],
                             idxbufs[slot], isem)

        def idx_wait(slot):
            pltpu.make_async_copy(ids_hbm.at[pl.ds(0, STEP_IROWS)],
                                  idxbufs[slot], isem).wait()

        def fire(slot):
            for j in range(STEP_IROWS):
                pltpu.async_copy(
                    table_hbm.at[idxbufs[slot].at[j]],
                    bufs[slot].at[pl.ds(j * IDXW, IDXW)],
                    sems[slot],
                )

        def drain(slot):
            pltpu.make_async_copy(
                table_hbm.at[pl.ds(0, ROWS_PER_STEP)], bufs[slot], sems[slot]
            ).wait()

        def accumulate(slot, irow0):
            buf = bufs[slot]
            for j in range(STEP_IROWS):
                lj = (irow0 + j) >> 7
                base = j * IDXW

                def body(r4, vaccs, base=base):
                    row = base + r4 * 4
                    out = []
                    for c in range(NCHUNK):
                        sl = pl.ds(c * 16, 16)
                        v01 = buf[row, sl] + buf[row + 1, sl]
                        v23 = buf[row + 2, sl] + buf[row + 3, sl]
                        out.append(vaccs[c] + (v01 + v23))
                    return tuple(out)

                zero = jnp.zeros((16,), jnp.float32)
                vaccs = lax.fori_loop(0, IDXW // 4, body,
                                      (zero, zero, zero, zero))
                for c in range(NCHUNK):
                    plsc.addupdate(acc.at[lj, pl.ds(c * 16, 16)], vaccs[c])

        idx_fetch(0, 0)
        idx_wait(0)
        fire(0)
        idx_fetch(1, 1)

        def phase(g, slot, nslot, fetch_ahead):
            idx_wait(nslot)
            fire(nslot)
            drain(slot)
            if fetch_ahead:
                idx_fetch(g + 2, slot)
            accumulate(slot, base_irow + g * STEP_IROWS)

        def main_body(g2, _):
            g = g2 * 2
            phase(g, 0, 1, True)
            phase(g + 1, 1, 0, True)
            return 0
        lax.fori_loop(0, nsteps // 2 - 1, main_body, 0)

        phase(nsteps - 2, 0, 1, False)
        drain(1)
        accumulate(1, base_irow + (nsteps - 1) * STEP_IROWS)

        pltpu.sync_copy(acc, out_hbm.at[wid])

    return k(ids2d, table)


def _mlp(partials, W1, b1, W2, b2, W3, b3):
    def body(p_ref, w1_ref, b1_ref, w2_ref, b2_ref, w3_ref, b3_ref, o_ref):
        s = jnp.sum(p_ref[...], axis=0) * (1.0 / B)
        h = jnp.maximum(
            jnp.dot(s, w1_ref[...], preferred_element_type=jnp.float32)
            + b1_ref[...], 0.0)
        h = jnp.maximum(
            jnp.dot(h, w2_ref[...], preferred_element_type=jnp.float32)
            + b2_ref[...], 0.0)
        o_ref[...] = (
            jnp.dot(h, w3_ref[...], preferred_element_type=jnp.float32)
            + b3_ref[...])

    return pl.pallas_call(
        body,
        out_shape=jax.ShapeDtypeStruct((L, D), jnp.float32),
    )(partials, W1, b1.reshape(1, D), W2, b2.reshape(1, D), W3,
      b3.reshape(1, D))


def kernel(tag_ids, table, W1, b1, W2, b2, W3, b3):
    ids2d = tag_ids.astype(jnp.int32).T.reshape(-1, IDXW)
    partials = _sc_partial_sums(ids2d, table)
    return _mlp(partials, W1, b1, W2, b2, W3, b3)
